# baseline (device time: 6650 ns/iter reference)
import jax
import jax.numpy as jnp
from jax import lax
from jax.experimental import pallas as pl
from jax.experimental.pallas import tpu as pltpu


def kernel(x):
    m, n = x.shape
    w = n // 2
    out_dtype = jnp.bfloat16

    def body(x_hbm, out_hbm, xp_vmem, xm_vmem, send_buf, out_local,
             cp_sem, cm_sem, co_sem, send_sem, recv_sem):
        my_x = lax.axis_index("x")
        my_y = lax.axis_index("y")
        my_z = lax.axis_index("z")
        peer_y = 1 - my_y
        peer = (my_x, peer_y, my_z)

        barrier = pltpu.get_barrier_semaphore()
        pl.semaphore_signal(
            barrier, inc=1,
            device_id=peer,
            device_id_type=pl.DeviceIdType.MESH,
        )

        cp = pltpu.make_async_copy(
            x_hbm.at[:, pl.ds(peer_y * w, w)], xp_vmem, cp_sem
        )
        cp.start()
        cm = pltpu.make_async_copy(
            x_hbm.at[:, pl.ds(my_y * w, w)], xm_vmem, cm_sem
        )
        cm.start()

        cp.wait()
        send_buf[...] = xp_vmem[...].astype(out_dtype)

        pl.semaphore_wait(barrier, 1)

        rdma = pltpu.make_async_remote_copy(
            src_ref=send_buf,
            dst_ref=out_hbm.at[pl.ds(my_y * m, m), :],
            send_sem=send_sem,
            recv_sem=recv_sem,
            device_id=peer,
            device_id_type=pl.DeviceIdType.MESH,
        )
        rdma.start()

        cm.wait()
        out_local[...] = xm_vmem[...].astype(out_dtype)
        co = pltpu.make_async_copy(
            out_local, out_hbm.at[pl.ds(my_y * m, m), :], co_sem
        )
        co.start()

        co.wait()
        rdma.wait()

    return pl.pallas_call(
        body,
        out_shape=jax.ShapeDtypeStruct((2 * m, w), out_dtype),
        in_specs=[pl.BlockSpec(memory_space=pl.ANY)],
        out_specs=pl.BlockSpec(memory_space=pl.ANY),
        scratch_shapes=[
            pltpu.VMEM((m, w), x.dtype),
            pltpu.VMEM((m, w), x.dtype),
            pltpu.VMEM((m, w), out_dtype),
            pltpu.VMEM((m, w), out_dtype),
            pltpu.SemaphoreType.DMA,
            pltpu.SemaphoreType.DMA,
            pltpu.SemaphoreType.DMA,
            pltpu.SemaphoreType.DMA,
            pltpu.SemaphoreType.DMA,
        ],
        compiler_params=pltpu.CompilerParams(collective_id=0),
    )(x)
